# baseline (speedup 1.0000x reference)
import jax
import jax.numpy as jnp
from jax.experimental import pallas as pl
from jax.experimental.pallas import tpu as pltpu

_D_MODEL = 4096
_E = 64
_K = 8
_T = 1024


def _gating_body(x_ref, w_ref, b_ref, gated_ref, weights_ref):
    x = x_ref[...]
    w = w_ref[...]
    b = b_ref[...]
    logits = jnp.dot(x, w, preferred_element_type=jnp.float32) + b
    lt = logits.T
    m = jnp.max(lt, axis=0, keepdims=True)
    e = jnp.exp(lt - m)
    s = jnp.sum(e, axis=0, keepdims=True)
    weights = e / s

    wv = weights
    for _ in range(_K):
        cur = jnp.max(wv, axis=0, keepdims=True)
        wv = jnp.where(wv == cur, -1.0, wv)
    gated_ref[...] = jnp.where(wv < 0, weights, 0.0)
    weights_ref[...] = weights


def kernel(x, W, b):
    B, S, D = x.shape
    N = B * S
    x2 = x.reshape(N, D)
    b2 = b.reshape(1, _E)
    grid = (N // _T,)
    out_shape = [
        jax.ShapeDtypeStruct((_E, N), jnp.float32),
        jax.ShapeDtypeStruct((_E, N), jnp.float32),
    ]
    gated, weights = pl.pallas_call(
        _gating_body,
        grid=grid,
        in_specs=[
            pl.BlockSpec((_T, D), lambda i: (i, 0)),
            pl.BlockSpec((D, _E), lambda i: (0, 0)),
            pl.BlockSpec((1, _E), lambda i: (0, 0)),
        ],
        out_specs=[
            pl.BlockSpec((_E, _T), lambda i: (0, i)),
            pl.BlockSpec((_E, _T), lambda i: (0, i)),
        ],
        out_shape=out_shape,
    )(x2, W, b2)
    return gated.reshape(_E, B, S), weights.reshape(_E, B, S)


# speedup vs baseline: 1.0014x; 1.0014x over previous
import jax
import jax.numpy as jnp
from jax.experimental import pallas as pl
from jax.experimental.pallas import tpu as pltpu

_D_MODEL = 4096
_E = 64
_K = 8
_T = 512
_DH = _D_MODEL // 2


def _gating_body(xa_ref, xb_ref, w_ref, b_ref, gated_ref, weights_ref):
    w = w_ref[...]
    b = b_ref[...]
    logits = (
        jnp.dot(xa_ref[...], w[:_DH], preferred_element_type=jnp.float32)
        + jnp.dot(xb_ref[...], w[_DH:], preferred_element_type=jnp.float32)
        + b
    )
    lt = logits.T
    m = jnp.max(lt, axis=0, keepdims=True)
    e = jnp.exp(lt - m)
    s = jnp.sum(e, axis=0, keepdims=True)
    weights = e / s

    wv = weights
    for _ in range(_K):
        cur = jnp.max(wv, axis=0, keepdims=True)
        wv = jnp.where(wv == cur, -1.0, wv)
    gated_ref[...] = jnp.where(wv < 0, weights, 0.0)
    weights_ref[...] = weights


def kernel(x, W, b):
    B, S, D = x.shape
    N = B * S
    x2 = x.reshape(N, D)
    b2 = b.reshape(1, _E)
    grid = (N // _T,)
    out_shape = [
        jax.ShapeDtypeStruct((_E, N), jnp.float32),
        jax.ShapeDtypeStruct((_E, N), jnp.float32),
    ]
    gated, weights = pl.pallas_call(
        _gating_body,
        grid=grid,
        in_specs=[
            pl.BlockSpec((_T, _DH), lambda i: (i, 0)),
            pl.BlockSpec((_T, _DH), lambda i: (i, 1)),
            pl.BlockSpec((D, _E), lambda i: (0, 0)),
            pl.BlockSpec((1, _E), lambda i: (0, 0)),
        ],
        out_specs=[
            pl.BlockSpec((_E, _T), lambda i: (0, i)),
            pl.BlockSpec((_E, _T), lambda i: (0, i)),
        ],
        out_shape=out_shape,
    )(x2, x2, W, b2)
    return gated.reshape(_E, B, S), weights.reshape(_E, B, S)


# speedup vs baseline: 1.0121x; 1.0107x over previous
import jax
import jax.numpy as jnp
from jax.experimental import pallas as pl

_E = 64
_K = 8
_T = 512


def _gating_body(x_ref, w_ref, b_ref, gated_ref, weights_ref):
    x = x_ref[...]
    w = w_ref[...]
    b = b_ref[...]
    logits = jnp.dot(x, w, preferred_element_type=jnp.float32) + b
    lt = logits.T
    m = jnp.max(lt, axis=0, keepdims=True)
    e = jnp.exp(lt - m)
    s = jnp.sum(e, axis=0, keepdims=True)
    weights = e / s

    wv = weights
    for _ in range(_K):
        cur = jnp.max(wv, axis=0, keepdims=True)
        wv = jnp.where(wv == cur, -1.0, wv)
    gated_ref[...] = jnp.where(wv < 0, weights, 0.0)
    weights_ref[...] = weights


def kernel(x, W, b):
    B, S, D = x.shape
    N = B * S
    x2 = x.reshape(N, D)
    b2 = b.reshape(1, _E)
    grid = (N // _T,)
    out_shape = [
        jax.ShapeDtypeStruct((_E, N), jnp.float32),
        jax.ShapeDtypeStruct((_E, N), jnp.float32),
    ]
    gated, weights = pl.pallas_call(
        _gating_body,
        grid=grid,
        in_specs=[
            pl.BlockSpec((_T, D), lambda i: (i, 0)),
            pl.BlockSpec((D, _E), lambda i: (0, 0)),
            pl.BlockSpec((1, _E), lambda i: (0, 0)),
        ],
        out_specs=[
            pl.BlockSpec((_E, _T), lambda i: (0, i)),
            pl.BlockSpec((_E, _T), lambda i: (0, i)),
        ],
        out_shape=out_shape,
    )(x2, W, b2)
    return gated.reshape(_E, B, S), weights.reshape(_E, B, S)
